# baseline (speedup 1.0000x reference)
import functools

import jax
import jax.numpy as jnp
from jax import lax
from jax.experimental import pallas as pl
from jax.experimental.pallas import tpu as pltpu
from jax.experimental.pallas import tpu_sc as plsc

B = 16384
F = 32
L = 16

_info = plsc.get_sparse_core_info()
NC, NS = _info.num_cores, _info.num_subcores
NW = NC * NS
B_PER_W = B // NW
CHUNK = 128
NCHUNK = B_PER_W // CHUNK


def _gmf_kernel(user_hbm, item_hbm, eu_hbm, ei_hbm, w_hbm, b_hbm, out_hbm,
                idx_u, idx_i, eu_v, ei_v, out_v, w_v, b_v, sem):
    wid = lax.axis_index("s") * NC + lax.axis_index("c")
    base = wid * B_PER_W

    pltpu.sync_copy(w_hbm, w_v)
    pltpu.sync_copy(b_hbm, b_v)

    for c in range(NCHUNK):
        pltpu.sync_copy(user_hbm.at[pl.ds(base + c * CHUNK, CHUNK)], idx_u.at[c])
        pltpu.sync_copy(item_hbm.at[pl.ds(base + c * CHUNK, CHUNK)], idx_i.at[c])

    copies = []
    for c in range(NCHUNK):
        copies.append(pltpu.async_copy(eu_hbm.at[idx_u.at[c]], eu_v.at[c], sem))
        copies.append(pltpu.async_copy(ei_hbm.at[idx_i.at[c]], ei_v.at[c], sem))
    for cp in copies:
        cp.wait()

    w_lo = w_v[0, pl.ds(0, L)]
    w_hi = w_v[0, pl.ds(L, L)]
    bias_v = b_v[...]
    lane = lax.iota(jnp.int32, L)

    def make_group_body(c):
        def group_body(g, carry):
            acc = jnp.zeros((L,), jnp.float32)
            for j in range(L):
                r = g * L + j
                eu_lo = eu_v[c, r, pl.ds(0, L)]
                eu_hi = eu_v[c, r, pl.ds(L, L)]
                ei_lo = ei_v[c, r, pl.ds(0, L)]
                ei_hi = ei_v[c, r, pl.ds(L, L)]
                t = (eu_lo * ei_lo) * w_lo + ((eu_hi * ei_hi) * w_hi + bias_v)
                acc = jnp.where(lane == j, jnp.sum(t), acc)
            out_v[pl.ds(c * CHUNK + g * L, L)] = acc
            return carry

        return group_body

    for c in range(NCHUNK):
        lax.fori_loop(0, CHUNK // L, make_group_body(c), 0)

    pltpu.sync_copy(out_v, out_hbm.at[pl.ds(base, B_PER_W)])


def kernel(user, item, embed_user, embed_item, W, b):
    mesh = plsc.VectorSubcoreMesh(core_axis_name="c", subcore_axis_name="s")
    run = pl.kernel(
        _gmf_kernel,
        mesh=mesh,
        compiler_params=pltpu.CompilerParams(
            needs_layout_passes=False, use_tc_tiling_on_sc=False
        ),
        out_type=jax.ShapeDtypeStruct((B,), jnp.float32),
        scratch_types=[
            pltpu.VMEM((NCHUNK, CHUNK), jnp.int32),
            pltpu.VMEM((NCHUNK, CHUNK), jnp.int32),
            pltpu.VMEM((NCHUNK, CHUNK, F), jnp.float32),
            pltpu.VMEM((NCHUNK, CHUNK, F), jnp.float32),
            pltpu.VMEM((B_PER_W,), jnp.float32),
            pltpu.VMEM((1, F), jnp.float32),
            pltpu.VMEM((L,), jnp.float32),
            pltpu.SemaphoreType.DMA,
        ],
    )
    b16 = jnp.pad(b.astype(jnp.float32), (0, L - 1))
    return run(user, item, embed_user, embed_item, W, b16)
